# baseline (speedup 1.0000x reference)
import functools

import jax
import jax.numpy as jnp
from jax import lax
from jax.experimental import pallas as pl
from jax.experimental.pallas import tpu as pltpu
from jax.experimental.pallas import tpu_sc as plsc

_NC = 2
_NS = 16
_NW = _NC * _NS
_L = 16

_LN2 = 0.6931471805599453
_SQRT2 = 1.4142135623730951


def _vlog(v):
    xi = lax.bitcast_convert_type(v, jnp.int32)
    e = ((xi >> 23) & 0xFF) - 127
    m = lax.bitcast_convert_type((xi & 0x7FFFFF) | 0x3F800000, jnp.float32)
    big = m > _SQRT2
    m = jnp.where(big, m * 0.5, m)
    e = jnp.where(big, e + 1, e)
    z = (m - 1.0) / (m + 1.0)
    z2 = z * z
    p = z * (2.0 + z2 * (2.0 / 3.0 + z2 * (0.4 + z2 * (2.0 / 7.0))))
    return e.astype(jnp.float32) * _LN2 + p


def _sc_body(rows_w, b_w, emb, v_rows,
             idx_hbm, x0_hbm, x1_hbm, xu0_hbm, xu1_hbm,
             pb_hbm, em2_hbm, es2_hbm, mgb_hbm, bias_t_hbm, ent_hbm,
             pred_out, klb_out, kle_out,
             idxv, kvalg, klev, bmv, bsv, klbv,
             xv0, xv1, cidx0, cidx1,
             e0g, e1g, b0m, b1m, predv,
             pbv, em2v, es2v, mgv,
             sem_k, sem_b, sem_c, sem_p, sem_o):
    ng = (2 * emb) // 8
    wid = lax.axis_index("s") * _NC + lax.axis_index("c")
    grp = wid // (_NW // 2)
    kbase = wid * rows_w
    pbase = wid * b_w
    iota = lax.iota(jnp.int32, _L)

    pltpu.sync_copy(idx_hbm.at[pl.ds(kbase, rows_w)], idxv)
    pltpu.sync_copy(x0_hbm.at[pl.ds(pbase, b_w)], xv0)
    pltpu.sync_copy(x1_hbm.at[pl.ds(pbase, b_w)], xv1)

    cb_m = pltpu.async_copy(bias_t_hbm.at[pl.ds(0, v_rows)].at[idxv], bmv, sem_b)
    cb_s = pltpu.async_copy(
        bias_t_hbm.at[pl.ds(v_rows, v_rows)].at[idxv], bsv, sem_b)
    cc0 = pltpu.async_copy(xu0_hbm.at[xv0], cidx0, sem_c)
    cc1 = pltpu.async_copy(xu1_hbm.at[xv1], cidx1, sem_c)

    kl_cps = []
    for j in range(ng):
        kl_cps.append(pltpu.async_copy(
            ent_hbm.at[j].at[idxv], kvalg.at[j], sem_k))

    pltpu.sync_copy(pb_hbm.at[grp], pbv)
    pltpu.sync_copy(em2_hbm.at[grp], em2v)
    pltpu.sync_copy(es2_hbm.at[grp], es2v)
    pltpu.sync_copy(mgb_hbm, mgv)

    cc0.wait()
    cc1.wait()
    p_cps = []
    p_cps.append(pltpu.async_copy(
        bias_t_hbm.at[pl.ds(0, v_rows)].at[cidx0], b0m, sem_p))
    p_cps.append(pltpu.async_copy(
        bias_t_hbm.at[pl.ds(0, v_rows)].at[cidx1], b1m, sem_p))
    for j in range(ng // 2):
        p_cps.append(pltpu.async_copy(
            ent_hbm.at[j].at[cidx0], e0g.at[j], sem_p))
        p_cps.append(pltpu.async_copy(
            ent_hbm.at[j].at[cidx1], e1g.at[j], sem_p))

    bm2 = pbv[0, :]
    bs2 = jnp.abs(pbv[1, :])
    log_bs2 = _vlog(bs2)
    inv2bs2 = 0.5 / (bs2 * bs2)
    cb_m.wait()
    cb_s.wait()

    def klb_body(j, carry):
        m1 = bmv[pl.ds(j * _L, _L)]
        s1 = jnp.abs(bsv[pl.ds(j * _L, _L)])
        d = m1 - bm2
        klbv[pl.ds(j * _L, _L)] = (
            log_bs2 - _vlog(s1) + (s1 * s1 + d * d) * inv2bs2 - 0.5)
        return carry

    lax.fori_loop(0, rows_w // _L, klb_body, 0)
    out_klb = pltpu.async_copy(klbv, klb_out.at[pl.ds(kbase, rows_w)], sem_o)

    for cp in kl_cps:
        cp.wait()

    out_kle = []
    for k in range(emb):
        jm, cm = k // 8, k % 8
        js, cs = (emb + k) // 8, (emb + k) % 8
        cmv = iota * 0 + cm
        csv = iota * 0 + cs
        em2k = em2v[k, :]
        es2k = jnp.abs(es2v[k, :])
        log_es2k = _vlog(es2k)
        inv2es2k = 0.5 / (es2k * es2k)

        def kle_body(j, carry, k=k, jm=jm, js=js, cmv=cmv, csv=csv,
                     em2k=em2k, log_es2k=log_es2k, inv2es2k=inv2es2k):
            r = j * _L + iota
            m1 = plsc.load_gather(kvalg.at[jm], [r, cmv])
            s1 = jnp.abs(plsc.load_gather(kvalg.at[js], [r, csv]))
            d = m1 - em2k
            klev[k, pl.ds(j * _L, _L)] = (
                log_es2k - _vlog(s1) + (s1 * s1 + d * d) * inv2es2k - 0.5)
            return carry

        lax.fori_loop(0, rows_w // _L, kle_body, 0)
        out_kle.append(pltpu.async_copy(
            klev.at[k], kle_out.at[k, pl.ds(kbase, rows_w)], sem_o))

    for cp in p_cps:
        cp.wait()
    mg = mgv[...]

    def pred_body(j, carry):
        sl = pl.ds(j * _L, _L)
        r = j * _L + iota
        acc = mg + b0m[sl] + b1m[sl]
        for k in range(emb):
            cv = iota * 0 + (k % 8)
            acc = acc + (plsc.load_gather(e0g.at[k // 8], [r, cv])
                         * plsc.load_gather(e1g.at[k // 8], [r, cv]))
        predv[sl] = acc
        return carry

    lax.fori_loop(0, b_w // _L, pred_body, 0)
    pltpu.sync_copy(predv, pred_out.at[pl.ds(pbase, b_w)])

    out_klb.wait()
    for cp in out_kle:
        cp.wait()


@jax.jit
def _cf_sc(idx_flat, x0, x1, xu0, xu1, pb, em2_bc, es2_bc, mgbv, bias_t,
           ent_g8):
    two_u = idx_flat.shape[0]
    b = x0.shape[0]
    emb = em2_bc.shape[1]
    ng = ent_g8.shape[0]
    v_rows = bias_t.shape[0] // 2
    rows_w = two_u // _NW
    b_w = b // _NW
    mesh = plsc.VectorSubcoreMesh(
        core_axis_name="c", subcore_axis_name="s", num_cores=_NC)
    body = functools.partial(_sc_body, rows_w, b_w, emb, v_rows)
    fn = pl.kernel(
        body,
        out_type=(
            jax.ShapeDtypeStruct((b,), jnp.float32),
            jax.ShapeDtypeStruct((two_u,), jnp.float32),
            jax.ShapeDtypeStruct((emb, two_u), jnp.float32),
        ),
        mesh=mesh,
        compiler_params=pltpu.CompilerParams(
            needs_layout_passes=False, use_tc_tiling_on_sc=False),
        scratch_types=[
            pltpu.VMEM((rows_w,), jnp.int32),
            pltpu.VMEM((ng, rows_w, 8), jnp.float32),
            pltpu.VMEM((emb, rows_w), jnp.float32),
            pltpu.VMEM((rows_w,), jnp.float32),
            pltpu.VMEM((rows_w,), jnp.float32),
            pltpu.VMEM((rows_w,), jnp.float32),
            pltpu.VMEM((b_w,), jnp.int32),
            pltpu.VMEM((b_w,), jnp.int32),
            pltpu.VMEM((b_w,), jnp.int32),
            pltpu.VMEM((b_w,), jnp.int32),
            pltpu.VMEM((ng // 2, b_w, 8), jnp.float32),
            pltpu.VMEM((ng // 2, b_w, 8), jnp.float32),
            pltpu.VMEM((b_w,), jnp.float32),
            pltpu.VMEM((b_w,), jnp.float32),
            pltpu.VMEM((b_w,), jnp.float32),
            pltpu.VMEM((2, _L), jnp.float32),
            pltpu.VMEM((_L, _L), jnp.float32),
            pltpu.VMEM((_L, _L), jnp.float32),
            pltpu.VMEM((_L,), jnp.float32),
            pltpu.SemaphoreType.DMA,
            pltpu.SemaphoreType.DMA,
            pltpu.SemaphoreType.DMA,
            pltpu.SemaphoreType.DMA,
            pltpu.SemaphoreType.DMA,
        ],
    )
    return fn(idx_flat, x0, x1, xu0, xu1, pb, em2_bc, es2_bc, mgbv, bias_t,
              ent_g8)


def kernel(x, x_unique, alpha, mean_global_bias, scale_global_bias,
           bias_params, entity_params, mean_global_bias_prior,
           scale_global_bias_prior, mean_group_bias_prior,
           scale_group_bias_prior, mean_group_entity_prior,
           scale_group_entity_prior):
    emb = entity_params.shape[1] // 2
    u = x_unique.shape[1]

    idx_flat = jnp.reshape(x_unique, (2 * u,))
    v = entity_params.shape[0]
    ent_t = jnp.transpose(entity_params.astype(jnp.float32))
    ent_g8 = jnp.transpose(jnp.reshape(ent_t, (2 * emb // 8, 8, v)),
                           (0, 2, 1))
    bias_t = jnp.reshape(jnp.transpose(bias_params.astype(jnp.float32)),
                         (-1,))

    pb = jnp.stack([
        jnp.broadcast_to(mean_group_bias_prior, (2, _L)),
        jnp.broadcast_to(scale_group_bias_prior, (2, _L)),
    ], axis=1).astype(jnp.float32)
    em2_bc = jnp.broadcast_to(
        mean_group_entity_prior.astype(jnp.float32)[:, :, None], (2, emb, _L))
    es2_bc = jnp.broadcast_to(
        scale_group_entity_prior.astype(jnp.float32)[:, :, None], (2, emb, _L))
    mgbv = jnp.broadcast_to(mean_global_bias, (_L,)).astype(jnp.float32)

    unscaled_pred, kl_bias, kle_t = _cf_sc(
        idx_flat, x[0], x[1], x_unique[0], x_unique[1], pb, em2_bc, es2_bc,
        mgbv, bias_t, ent_g8)
    kl_entity = jnp.transpose(kle_t)

    std_dev = jnp.sqrt(1.0 / jnp.abs(alpha))
    s1g = jnp.abs(scale_global_bias)
    s2g = jnp.abs(scale_global_bias_prior)
    kl_global = (jnp.log(s2g / s1g)
                 + (s1g ** 2 + (mean_global_bias - mean_global_bias_prior) ** 2)
                 / (2.0 * s2g ** 2) - 0.5)
    return (unscaled_pred, std_dev, kl_global, kl_bias, kl_entity)


# speedup vs baseline: 3.1774x; 3.1774x over previous
import functools

import jax
import jax.numpy as jnp
from jax import lax
from jax.experimental import pallas as pl
from jax.experimental.pallas import tpu as pltpu
from jax.experimental.pallas import tpu_sc as plsc

_NC = 2
_NS = 16
_NW = _NC * _NS
_L = 16

_LN2 = 0.6931471805599453
_SQRT2 = 1.4142135623730951
_LC = (3.3401162467805445e-08, 1.0000030976858774, -0.500012926629249,
       0.33304818290092836, -0.24911238582368764, 0.2061172902526956,
       -0.18627400747929512, 0.11448230408724869)


def _vlog(v):
    xi = lax.bitcast_convert_type(v, jnp.int32)
    e = ((xi >> 23) & 0xFF) - 127
    m = lax.bitcast_convert_type((xi & 0x7FFFFF) | 0x3F800000, jnp.float32)
    big = m > _SQRT2
    m = jnp.where(big, m * 0.5, m)
    e = jnp.where(big, e + 1, e)
    w = m - 1.0
    w2 = w * w
    w4 = w2 * w2
    ab = (_LC[0] + _LC[1] * w) + (_LC[2] + _LC[3] * w) * w2
    cd = (_LC[4] + _LC[5] * w) + (_LC[6] + _LC[7] * w) * w2
    return e.astype(jnp.float32) * _LN2 + (ab + cd * w4)


def _sc_body(rows_w, b_w, emb, v_rows,
             idx_hbm, x0_hbm, x1_hbm, xu0_hbm, xu1_hbm,
             pb_hbm, em2_hbm, es2_hbm, mgb_hbm, bias_t_hbm, ent_hbm,
             pred_out, klb_out, kle_out,
             idxv, kval, bmv, bsv, klbv,
             xv0, xv1, cidx0, cidx1,
             e0t, e1t, b0m, b1m, predv,
             pbv, em2v, es2v, mgv,
             sem_k, sem_b, sem_c, sem_p, sem_o):
    wid = lax.axis_index("s") * _NC + lax.axis_index("c")
    grp = wid // (_NW // 2)
    kbase = wid * rows_w
    pbase = wid * b_w

    pltpu.sync_copy(idx_hbm.at[pl.ds(kbase, rows_w)], idxv)
    pltpu.sync_copy(x0_hbm.at[pl.ds(pbase, b_w)], xv0)
    pltpu.sync_copy(x1_hbm.at[pl.ds(pbase, b_w)], xv1)

    cb_m = pltpu.async_copy(bias_t_hbm.at[pl.ds(0, v_rows)].at[idxv], bmv, sem_b)
    cb_s = pltpu.async_copy(
        bias_t_hbm.at[pl.ds(v_rows, v_rows)].at[idxv], bsv, sem_b)
    cc0 = pltpu.async_copy(xu0_hbm.at[xv0], cidx0, sem_c)
    cc1 = pltpu.async_copy(xu1_hbm.at[xv1], cidx1, sem_c)

    kl_cps = []
    for k in range(emb):
        kl_cps.append(pltpu.async_copy(
            ent_hbm.at[pl.ds(k * v_rows, v_rows)].at[idxv], kval.at[k], sem_k))
        kl_cps.append(pltpu.async_copy(
            ent_hbm.at[pl.ds((emb + k) * v_rows, v_rows)].at[idxv],
            kval.at[emb + k], sem_k))

    pltpu.sync_copy(pb_hbm.at[grp], pbv)
    pltpu.sync_copy(em2_hbm.at[grp], em2v)
    pltpu.sync_copy(es2_hbm.at[grp], es2v)
    pltpu.sync_copy(mgb_hbm, mgv)

    cc0.wait()
    cc1.wait()
    p_cps = []
    p_cps.append(pltpu.async_copy(
        bias_t_hbm.at[pl.ds(0, v_rows)].at[cidx0], b0m, sem_p))
    p_cps.append(pltpu.async_copy(
        bias_t_hbm.at[pl.ds(0, v_rows)].at[cidx1], b1m, sem_p))
    for k in range(emb):
        p_cps.append(pltpu.async_copy(
            ent_hbm.at[pl.ds(k * v_rows, v_rows)].at[cidx0], e0t.at[k], sem_p))
        p_cps.append(pltpu.async_copy(
            ent_hbm.at[pl.ds(k * v_rows, v_rows)].at[cidx1], e1t.at[k], sem_p))

    bm2 = pbv[0, :]
    bs2 = jnp.abs(pbv[1, :])
    log_bs2 = _vlog(bs2)
    inv2bs2 = 0.5 / (bs2 * bs2)
    cb_m.wait()
    cb_s.wait()

    def klb_body(j, carry):
        m1 = bmv[pl.ds(j * _L, _L)]
        s1 = jnp.abs(bsv[pl.ds(j * _L, _L)])
        d = m1 - bm2
        klbv[pl.ds(j * _L, _L)] = (
            log_bs2 - _vlog(s1) + (s1 * s1 + d * d) * inv2bs2 - 0.5)
        return carry

    lax.fori_loop(0, rows_w // _L, klb_body, 0)
    out_klb = pltpu.async_copy(klbv, klb_out.at[pl.ds(kbase, rows_w)], sem_o)

    out_kle = []
    for k in range(emb):
        kl_cps[2 * k].wait()
        kl_cps[2 * k + 1].wait()
        em2k = em2v[k, :]
        es2k = jnp.abs(es2v[k, :])
        log_es2k = _vlog(es2k)
        inv2es2k = 0.5 / (es2k * es2k)

        def kle_body(j, carry, k=k, em2k=em2k, log_es2k=log_es2k,
                     inv2es2k=inv2es2k):
            m1 = kval[k, pl.ds(j * _L, _L)]
            s1 = jnp.abs(kval[emb + k, pl.ds(j * _L, _L)])
            d = m1 - em2k
            kval[k, pl.ds(j * _L, _L)] = (
                log_es2k - _vlog(s1) + (s1 * s1 + d * d) * inv2es2k - 0.5)
            return carry

        lax.fori_loop(0, rows_w // _L, kle_body, 0)
        out_kle.append(pltpu.async_copy(
            kval.at[k], kle_out.at[k, pl.ds(kbase, rows_w)], sem_o))

    for cp in p_cps:
        cp.wait()
    mg = mgv[...]

    def pred_body(j, carry):
        sl = pl.ds(j * _L, _L)
        acc = mg + b0m[sl] + b1m[sl]
        for k in range(emb):
            acc = acc + e0t[k, sl] * e1t[k, sl]
        predv[sl] = acc
        return carry

    lax.fori_loop(0, b_w // _L, pred_body, 0)
    pltpu.sync_copy(predv, pred_out.at[pl.ds(pbase, b_w)])

    out_klb.wait()
    for cp in out_kle:
        cp.wait()


@jax.jit
def _cf_sc(idx_flat, x0, x1, xu0, xu1, pb, em2_bc, es2_bc, mgbv, bias_t,
           ent_flat):
    two_u = idx_flat.shape[0]
    b = x0.shape[0]
    emb = em2_bc.shape[1]
    v_rows = bias_t.shape[0] // 2
    rows_w = two_u // _NW
    b_w = b // _NW
    mesh = plsc.VectorSubcoreMesh(
        core_axis_name="c", subcore_axis_name="s", num_cores=_NC)
    body = functools.partial(_sc_body, rows_w, b_w, emb, v_rows)
    fn = pl.kernel(
        body,
        out_type=(
            jax.ShapeDtypeStruct((b,), jnp.float32),
            jax.ShapeDtypeStruct((two_u,), jnp.float32),
            jax.ShapeDtypeStruct((emb, two_u), jnp.float32),
        ),
        mesh=mesh,
        compiler_params=pltpu.CompilerParams(
            needs_layout_passes=False, use_tc_tiling_on_sc=False),
        scratch_types=[
            pltpu.VMEM((rows_w,), jnp.int32),
            pltpu.VMEM((2 * emb, rows_w), jnp.float32),
            pltpu.VMEM((rows_w,), jnp.float32),
            pltpu.VMEM((rows_w,), jnp.float32),
            pltpu.VMEM((rows_w,), jnp.float32),
            pltpu.VMEM((b_w,), jnp.int32),
            pltpu.VMEM((b_w,), jnp.int32),
            pltpu.VMEM((b_w,), jnp.int32),
            pltpu.VMEM((b_w,), jnp.int32),
            pltpu.VMEM((emb, b_w), jnp.float32),
            pltpu.VMEM((emb, b_w), jnp.float32),
            pltpu.VMEM((b_w,), jnp.float32),
            pltpu.VMEM((b_w,), jnp.float32),
            pltpu.VMEM((b_w,), jnp.float32),
            pltpu.VMEM((2, _L), jnp.float32),
            pltpu.VMEM((_L, _L), jnp.float32),
            pltpu.VMEM((_L, _L), jnp.float32),
            pltpu.VMEM((_L,), jnp.float32),
            pltpu.SemaphoreType.DMA,
            pltpu.SemaphoreType.DMA,
            pltpu.SemaphoreType.DMA,
            pltpu.SemaphoreType.DMA,
            pltpu.SemaphoreType.DMA,
        ],
    )
    return fn(idx_flat, x0, x1, xu0, xu1, pb, em2_bc, es2_bc, mgbv, bias_t,
              ent_flat)


def kernel(x, x_unique, alpha, mean_global_bias, scale_global_bias,
           bias_params, entity_params, mean_global_bias_prior,
           scale_global_bias_prior, mean_group_bias_prior,
           scale_group_bias_prior, mean_group_entity_prior,
           scale_group_entity_prior):
    emb = entity_params.shape[1] // 2
    u = x_unique.shape[1]

    idx_flat = jnp.reshape(x_unique, (2 * u,))
    ent_flat = jnp.reshape(jnp.transpose(entity_params.astype(jnp.float32)),
                           (-1,))
    bias_t = jnp.reshape(jnp.transpose(bias_params.astype(jnp.float32)),
                         (-1,))

    pb = jnp.stack([
        jnp.broadcast_to(mean_group_bias_prior, (2, _L)),
        jnp.broadcast_to(scale_group_bias_prior, (2, _L)),
    ], axis=1).astype(jnp.float32)
    em2_bc = jnp.broadcast_to(
        mean_group_entity_prior.astype(jnp.float32)[:, :, None], (2, emb, _L))
    es2_bc = jnp.broadcast_to(
        scale_group_entity_prior.astype(jnp.float32)[:, :, None], (2, emb, _L))
    mgbv = jnp.broadcast_to(mean_global_bias, (_L,)).astype(jnp.float32)

    unscaled_pred, kl_bias, kle_t = _cf_sc(
        idx_flat, x[0], x[1], x_unique[0], x_unique[1], pb, em2_bc, es2_bc,
        mgbv, bias_t, ent_flat)
    kl_entity = jnp.transpose(kle_t)

    std_dev = jnp.sqrt(1.0 / jnp.abs(alpha))
    s1g = jnp.abs(scale_global_bias)
    s2g = jnp.abs(scale_global_bias_prior)
    kl_global = (jnp.log(s2g / s1g)
                 + (s1g ** 2 + (mean_global_bias - mean_global_bias_prior) ** 2)
                 / (2.0 * s2g ** 2) - 0.5)
    return (unscaled_pred, std_dev, kl_global, kl_bias, kl_entity)


# speedup vs baseline: 3.2512x; 1.0232x over previous
import functools

import jax
import jax.numpy as jnp
from jax import lax
from jax.experimental import pallas as pl
from jax.experimental.pallas import tpu as pltpu
from jax.experimental.pallas import tpu_sc as plsc

_NC = 2
_NS = 16
_NW = _NC * _NS
_L = 16

_LN2 = 0.6931471805599453
_SQRT2 = 1.4142135623730951
_LC = (3.3401162467805445e-08, 1.0000030976858774, -0.500012926629249,
       0.33304818290092836, -0.24911238582368764, 0.2061172902526956,
       -0.18627400747929512, 0.11448230408724869)


def _vlog(v):
    xi = lax.bitcast_convert_type(v, jnp.int32)
    e = ((xi >> 23) & 0xFF) - 127
    m = lax.bitcast_convert_type((xi & 0x7FFFFF) | 0x3F800000, jnp.float32)
    big = m > _SQRT2
    m = jnp.where(big, m * 0.5, m)
    e = jnp.where(big, e + 1, e)
    w = m - 1.0
    w2 = w * w
    w4 = w2 * w2
    ab = (_LC[0] + _LC[1] * w) + (_LC[2] + _LC[3] * w) * w2
    cd = (_LC[4] + _LC[5] * w) + (_LC[6] + _LC[7] * w) * w2
    return e.astype(jnp.float32) * _LN2 + (ab + cd * w4)


def _sc_body(rows_w, b_w, emb, v_rows,
             idx_hbm, x0_hbm, x1_hbm, xu0_hbm, xu1_hbm,
             pb_hbm, em2_hbm, es2_hbm, mgb_hbm, bias_t_hbm, ent_hbm,
             pred_out, klb_out, kle_out,
             idxv, kval, bmv, bsv, klbv,
             xv0, xv1, cidx0, cidx1,
             e0t, e1t, b0m, b1m, predv,
             pbv, em2v, es2v, mgv,
             sem_k, sem_b, sem_c, sem_p, sem_o):
    wid = lax.axis_index("s") * _NC + lax.axis_index("c")
    grp = wid // (_NW // 2)
    kbase = wid * rows_w
    pbase = wid * b_w

    pltpu.sync_copy(idx_hbm.at[pl.ds(kbase, rows_w)], idxv)
    pltpu.sync_copy(x0_hbm.at[pl.ds(pbase, b_w)], xv0)
    pltpu.sync_copy(x1_hbm.at[pl.ds(pbase, b_w)], xv1)

    cb_m = pltpu.async_copy(bias_t_hbm.at[pl.ds(0, v_rows)].at[idxv], bmv, sem_b)
    cb_s = pltpu.async_copy(
        bias_t_hbm.at[pl.ds(v_rows, v_rows)].at[idxv], bsv, sem_b)
    cc0 = pltpu.async_copy(xu0_hbm.at[xv0], cidx0, sem_c)
    cc1 = pltpu.async_copy(xu1_hbm.at[xv1], cidx1, sem_c)

    kl_cps = []
    for k in range(emb):
        kl_cps.append(pltpu.async_copy(
            ent_hbm.at[pl.ds(k * v_rows, v_rows)].at[idxv], kval.at[k], sem_k))
        kl_cps.append(pltpu.async_copy(
            ent_hbm.at[pl.ds((emb + k) * v_rows, v_rows)].at[idxv],
            kval.at[emb + k], sem_k))

    pltpu.sync_copy(pb_hbm.at[grp], pbv)
    pltpu.sync_copy(em2_hbm.at[grp], em2v)
    pltpu.sync_copy(es2_hbm.at[grp], es2v)
    pltpu.sync_copy(mgb_hbm, mgv)

    cc0.wait()
    cc1.wait()
    p_cps = []
    p_cps.append(pltpu.async_copy(
        bias_t_hbm.at[pl.ds(0, v_rows)].at[cidx0], b0m, sem_p))
    p_cps.append(pltpu.async_copy(
        bias_t_hbm.at[pl.ds(0, v_rows)].at[cidx1], b1m, sem_p))
    for k in range(emb):
        p_cps.append(pltpu.async_copy(
            ent_hbm.at[pl.ds(k * v_rows, v_rows)].at[cidx0], e0t.at[k], sem_p))
        p_cps.append(pltpu.async_copy(
            ent_hbm.at[pl.ds(k * v_rows, v_rows)].at[cidx1], e1t.at[k], sem_p))

    bm2 = pbv[0, :]
    bs2 = jnp.abs(pbv[1, :])
    log_bs2 = _vlog(bs2)
    inv2bs2 = 0.5 / (bs2 * bs2)
    cb_m.wait()
    cb_s.wait()

    def klb_body(j, carry):
        m1 = bmv[pl.ds(j * _L, _L)]
        s1 = jnp.abs(bsv[pl.ds(j * _L, _L)])
        d = m1 - bm2
        klbv[pl.ds(j * _L, _L)] = (
            log_bs2 - _vlog(s1) + (s1 * s1 + d * d) * inv2bs2 - 0.5)
        return carry

    lax.fori_loop(0, rows_w // _L, klb_body, 0)
    out_klb = pltpu.async_copy(klbv, klb_out.at[pl.ds(kbase, rows_w)], sem_o)

    out_kle = []
    for k in range(emb):
        kl_cps[2 * k].wait()
        kl_cps[2 * k + 1].wait()
        em2k = em2v[k, :]
        es2k = jnp.abs(es2v[k, :])
        ck = _vlog(es2k) - 0.5
        inv2es2k = 0.5 / (es2k * es2k)

        def kle_body(j, carry, k=k, em2k=em2k, ck=ck, inv2es2k=inv2es2k):
            for h in range(2):
                sl = pl.ds(j * 2 * _L + h * _L, _L)
                m1 = kval[k, sl]
                s1 = jnp.abs(kval[emb + k, sl])
                d = m1 - em2k
                kval[k, sl] = (
                    ck - _vlog(s1) + (s1 * s1 + d * d) * inv2es2k)
            return carry

        lax.fori_loop(0, rows_w // (2 * _L), kle_body, 0)
        out_kle.append(pltpu.async_copy(
            kval.at[k], kle_out.at[k, pl.ds(kbase, rows_w)], sem_o))

    for cp in p_cps:
        cp.wait()
    mg = mgv[...]

    def pred_body(j, carry):
        sl = pl.ds(j * _L, _L)
        acc = mg + b0m[sl] + b1m[sl]
        for k in range(emb):
            acc = acc + e0t[k, sl] * e1t[k, sl]
        predv[sl] = acc
        return carry

    lax.fori_loop(0, b_w // _L, pred_body, 0)
    pltpu.sync_copy(predv, pred_out.at[pl.ds(pbase, b_w)])

    out_klb.wait()
    for cp in out_kle:
        cp.wait()


@jax.jit
def _cf_sc(idx_flat, x0, x1, xu0, xu1, pb, em2_bc, es2_bc, mgbv, bias_t,
           ent_flat):
    two_u = idx_flat.shape[0]
    b = x0.shape[0]
    emb = em2_bc.shape[1]
    v_rows = bias_t.shape[0] // 2
    rows_w = two_u // _NW
    b_w = b // _NW
    mesh = plsc.VectorSubcoreMesh(
        core_axis_name="c", subcore_axis_name="s", num_cores=_NC)
    body = functools.partial(_sc_body, rows_w, b_w, emb, v_rows)
    fn = pl.kernel(
        body,
        out_type=(
            jax.ShapeDtypeStruct((b,), jnp.float32),
            jax.ShapeDtypeStruct((two_u,), jnp.float32),
            jax.ShapeDtypeStruct((emb, two_u), jnp.float32),
        ),
        mesh=mesh,
        compiler_params=pltpu.CompilerParams(
            needs_layout_passes=False, use_tc_tiling_on_sc=False),
        scratch_types=[
            pltpu.VMEM((rows_w,), jnp.int32),
            pltpu.VMEM((2 * emb, rows_w), jnp.float32),
            pltpu.VMEM((rows_w,), jnp.float32),
            pltpu.VMEM((rows_w,), jnp.float32),
            pltpu.VMEM((rows_w,), jnp.float32),
            pltpu.VMEM((b_w,), jnp.int32),
            pltpu.VMEM((b_w,), jnp.int32),
            pltpu.VMEM((b_w,), jnp.int32),
            pltpu.VMEM((b_w,), jnp.int32),
            pltpu.VMEM((emb, b_w), jnp.float32),
            pltpu.VMEM((emb, b_w), jnp.float32),
            pltpu.VMEM((b_w,), jnp.float32),
            pltpu.VMEM((b_w,), jnp.float32),
            pltpu.VMEM((b_w,), jnp.float32),
            pltpu.VMEM((2, _L), jnp.float32),
            pltpu.VMEM((_L, _L), jnp.float32),
            pltpu.VMEM((_L, _L), jnp.float32),
            pltpu.VMEM((_L,), jnp.float32),
            pltpu.SemaphoreType.DMA,
            pltpu.SemaphoreType.DMA,
            pltpu.SemaphoreType.DMA,
            pltpu.SemaphoreType.DMA,
            pltpu.SemaphoreType.DMA,
        ],
    )
    return fn(idx_flat, x0, x1, xu0, xu1, pb, em2_bc, es2_bc, mgbv, bias_t,
              ent_flat)


def kernel(x, x_unique, alpha, mean_global_bias, scale_global_bias,
           bias_params, entity_params, mean_global_bias_prior,
           scale_global_bias_prior, mean_group_bias_prior,
           scale_group_bias_prior, mean_group_entity_prior,
           scale_group_entity_prior):
    emb = entity_params.shape[1] // 2
    u = x_unique.shape[1]

    idx_flat = jnp.reshape(x_unique, (2 * u,))
    ent_flat = jnp.reshape(jnp.transpose(entity_params.astype(jnp.float32)),
                           (-1,))
    bias_t = jnp.reshape(jnp.transpose(bias_params.astype(jnp.float32)),
                         (-1,))

    pb = jnp.stack([
        jnp.broadcast_to(mean_group_bias_prior, (2, _L)),
        jnp.broadcast_to(scale_group_bias_prior, (2, _L)),
    ], axis=1).astype(jnp.float32)
    em2_bc = jnp.broadcast_to(
        mean_group_entity_prior.astype(jnp.float32)[:, :, None], (2, emb, _L))
    es2_bc = jnp.broadcast_to(
        scale_group_entity_prior.astype(jnp.float32)[:, :, None], (2, emb, _L))
    mgbv = jnp.broadcast_to(mean_global_bias, (_L,)).astype(jnp.float32)

    unscaled_pred, kl_bias, kle_t = _cf_sc(
        idx_flat, x[0], x[1], x_unique[0], x_unique[1], pb, em2_bc, es2_bc,
        mgbv, bias_t, ent_flat)
    kl_entity = jnp.transpose(kle_t)

    std_dev = jnp.sqrt(1.0 / jnp.abs(alpha))
    s1g = jnp.abs(scale_global_bias)
    s2g = jnp.abs(scale_global_bias_prior)
    kl_global = (jnp.log(s2g / s1g)
                 + (s1g ** 2 + (mean_global_bias - mean_global_bias_prior) ** 2)
                 / (2.0 * s2g ** 2) - 0.5)
    return (unscaled_pred, std_dev, kl_global, kl_bias, kl_entity)
